# baseline (speedup 1.0000x reference)
import jax
import jax.numpy as jnp
from jax import lax
from jax.experimental import pallas as pl
from jax.experimental.pallas import tpu as pltpu
from jax.experimental.pallas import tpu_sc as plsc

N = 10000
D = 128
H = 128
OUT = 128
L = 3

NC = 2
NS = 16
NW = NC * NS
CHUNK = 128
NCHUNK = 80
NB = 2
SEG = 8
CTOT = 160
CA = 80
CB = CTOT - CA
EPW = NCHUNK * CHUNK
EPAD = EPW * NW
NPAD = 10240
RPT = NPAD // NS
BR = 256


def _mesh():
    return plsc.VectorSubcoreMesh(core_axis_name="c", subcore_axis_name="s")


def _sc_degree_body(dst_hbm, out_hbm, dst_v, ones_v, deg_sh):
    cid = lax.axis_index("c")
    sid = lax.axis_index("s")
    wid = sid * NC + cid
    base = sid * RPT

    def zrow(r, _):
        ones_v[r] = jnp.zeros((16,), jnp.float32)
        return 0
    lax.fori_loop(0, CHUNK, zrow, 0)
    for k in range(RPT // CHUNK):
        pltpu.sync_copy(ones_v, deg_sh.at[pl.ds(base + k * CHUNK, CHUNK)])

    def orow(r, _):
        ones_v[r] = jnp.ones((16,), jnp.float32)
        return 0
    lax.fori_loop(0, CHUNK, orow, 0)
    plsc.subcore_barrier()

    pltpu.sync_copy(dst_hbm.at[wid], dst_v)

    def step(j, _):
        pltpu.sync_copy(ones_v, deg_sh.at[dst_v.at[j]], add=True)
        return 0
    lax.fori_loop(0, NCHUNK, step, 0)

    plsc.subcore_barrier()
    pltpu.sync_copy(deg_sh.at[pl.ds(base, RPT)], out_hbm.at[cid, pl.ds(base, RPT)])


def _sc_degree(dst3):
    return pl.kernel(
        _sc_degree_body,
        out_type=jax.ShapeDtypeStruct((NC, NPAD, 16), jnp.float32),
        mesh=_mesh(),
        scratch_types=[
            pltpu.VMEM((NCHUNK, CHUNK), jnp.int32),
            pltpu.VMEM((CHUNK, 16), jnp.float32),
            pltpu.VMEM_SHARED((NPAD, 16), jnp.float32),
        ],
    )(dst3)


def _sc_edges_body(g_hbm, src_hbm, dst_hbm, out_hbm, src_v, dst_v,
                   r0, acc_sh, s0):
    cid = lax.axis_index("c")
    sid = lax.axis_index("s")
    base = sid * RPT

    def zrow(r, _):
        def zcol(cc, _):
            r0[r, pl.ds(cc * 16, 16)] = jnp.zeros((16,), jnp.float32)
            return 0
        lax.fori_loop(0, H // 16, zcol, 0)
        return 0
    lax.fori_loop(0, CHUNK, zrow, 0)
    for k in range(RPT // CHUNK):
        pltpu.sync_copy(r0, acc_sh.at[pl.ds(base + k * CHUNK, CHUNK)])
    plsc.subcore_barrier()

    def chunk_body(j, _):
        pltpu.async_copy(g_hbm.at[src_v.at[j]], r0, s0).wait()
        pltpu.sync_copy(r0, acc_sh.at[dst_v.at[j]], add=True)
        return 0

    wid = sid * NC + cid
    pltpu.sync_copy(src_hbm.at[wid], src_v)
    pltpu.sync_copy(dst_hbm.at[wid], dst_v)

    lax.fori_loop(0, CA, chunk_body, 0)

    plsc.subcore_barrier()
    for k in range(RPT // CHUNK):
        pltpu.sync_copy(
            acc_sh.at[pl.ds(base + k * CHUNK, CHUNK)],
            out_hbm.at[cid, pl.ds(base + k * CHUNK, CHUNK)],
        )


def _sc_edges(g, src4, dst4):
    return pl.kernel(
        _sc_edges_body,
        out_type=jax.ShapeDtypeStruct((NC, NPAD, H), jnp.float32),
        mesh=_mesh(),
        scratch_types=[
            pltpu.VMEM((CA, CHUNK), jnp.int32),
            pltpu.VMEM((CA, CHUNK), jnp.int32),
            pltpu.VMEM((CHUNK, H), jnp.float32),
            pltpu.VMEM_SHARED((NPAD, H), jnp.float32),
            pltpu.SemaphoreType.DMA,
        ],
    )(g, src4, dst4)


def _tc_prep_body(x_ref, wt_ref, deg_ref, g_ref):
    h = jnp.dot(x_ref[...], wt_ref[...], preferred_element_type=jnp.float32)
    d = deg_ref[0] + deg_ref[1]
    dinv = lax.rsqrt(d[:, 0:1] + 1.0)
    g_ref[...] = h * dinv


def _tc_prep(x_p, wt, deg2):
    grid = (NPAD // BR,)
    return pl.pallas_call(
        _tc_prep_body,
        grid=grid,
        in_specs=[
            pl.BlockSpec((BR, D), lambda i: (i, 0)),
            pl.BlockSpec((D, H), lambda i: (0, 0)),
            pl.BlockSpec((NC, BR, 16), lambda i: (0, i, 0)),
        ],
        out_specs=pl.BlockSpec((BR, H), lambda i: (i, 0)),
        out_shape=jax.ShapeDtypeStruct((NPAD, H), jnp.float32),
    )(x_p, wt, deg2)


def _tc_final_body(acc_ref, g_ref, deg_ref, gcnb_ref, wih_ref, bih_ref,
                   bhh_ref, linwt_ref, linb_ref, out_ref):
    d = deg_ref[0] + deg_ref[1]
    dinv = lax.rsqrt(d[:, 0:1] + 1.0)
    a = acc_ref[0] + acc_ref[1] + g_ref[...]
    z = jnp.maximum(a * dinv + gcnb_ref[...], 0.0)
    for l in range(L):
        gates = jnp.dot(z.astype(jnp.bfloat16), wih_ref[l],
                        preferred_element_type=jnp.float32)
        gates = gates + bih_ref[l] + bhh_ref[l]
        i_ = gates[:, 0:H]
        gg = gates[:, 2 * H:3 * H]
        o_ = gates[:, 3 * H:4 * H]
        c = jax.nn.sigmoid(i_) * jnp.tanh(gg)
        z = jax.nn.sigmoid(o_) * jnp.tanh(c)
    out_ref[...] = (
        jnp.dot(z.astype(jnp.bfloat16), linwt_ref[...],
                preferred_element_type=jnp.float32)
        + linb_ref[...]
    )


def _tc_final(acc2, g, deg2, gcn_b2, wih_t, b_ih, b_hh, lin_wt, lin_b2):
    grid = (NPAD // BR,)
    return pl.pallas_call(
        _tc_final_body,
        grid=grid,
        in_specs=[
            pl.BlockSpec((NC, BR, H), lambda i: (0, i, 0)),
            pl.BlockSpec((BR, H), lambda i: (i, 0)),
            pl.BlockSpec((NC, BR, 16), lambda i: (0, i, 0)),
            pl.BlockSpec((1, H), lambda i: (0, 0)),
            pl.BlockSpec((L, H, 4 * H), lambda i: (0, 0, 0)),
            pl.BlockSpec((L, 4 * H), lambda i: (0, 0)),
            pl.BlockSpec((L, 4 * H), lambda i: (0, 0)),
            pl.BlockSpec((H, OUT), lambda i: (0, 0)),
            pl.BlockSpec((1, OUT), lambda i: (0, 0)),
        ],
        out_specs=pl.BlockSpec((BR, OUT), lambda i: (i, 0)),
        out_shape=jax.ShapeDtypeStruct((NPAD, OUT), jnp.float32),
    )(acc2, g, deg2, gcn_b2, wih_t, b_ih, b_hh, lin_wt, lin_b2)


@jax.jit
def kernel(x, edge_index, gcn_W, gcn_b, W_ih, W_hh, b_ih, b_hh, lin_W, lin_b):
    del W_hh

    x_p = jnp.zeros((NPAD, D), jnp.float32).at[:N].set(x)
    src = edge_index[0]
    dst = edge_index[1]
    pad = EPAD - src.shape[0]
    srcp = jnp.concatenate([src, jnp.zeros((pad,), jnp.int32)])
    dstp = jnp.concatenate([dst, jnp.full((pad,), N, jnp.int32)])
    dst3 = dstp.reshape(NW, NCHUNK, CHUNK)
    def _regions(flat, fill):
        f3 = flat.reshape(NS, CTOT, CHUNK)
        b_pad = jnp.full((NS, CA - CB, CHUNK), fill, jnp.int32)
        r = jnp.stack(
            [f3[:, :CA], jnp.concatenate([f3[:, CA:], b_pad], axis=1)],
            axis=1)
        return r.reshape(NS * NC, CA, CHUNK)
    src4 = _regions(srcp, 0)
    dst4 = _regions(dstp, N)
    wt = gcn_W.T
    wih_t = W_ih.transpose(0, 2, 1).astype(jnp.bfloat16)
    lin_wt = lin_W.T.astype(jnp.bfloat16)
    gcn_b2 = gcn_b.reshape(1, H)
    lin_b2 = lin_b.reshape(1, OUT)

    deg2 = _sc_degree(dst3)
    g = _tc_prep(x_p, wt, deg2)
    acc2 = _sc_edges(g, src4, dst4)
    out = _tc_final(acc2, g, deg2, gcn_b2, wih_t, b_ih, b_hh, lin_wt,
                    lin_b2)
    return out[:N]


# speedup vs baseline: 1.5412x; 1.5412x over previous
import jax
import jax.numpy as jnp
from jax import lax
from jax.experimental import pallas as pl
from jax.experimental.pallas import tpu as pltpu
from jax.experimental.pallas import tpu_sc as plsc

N = 10000
D = 128
H = 128
OUT = 128
L = 3

NC = 2
NS = 16
NW = NC * NS
CHUNK = 128
NCHUNK = 79
EPW = NCHUNK * CHUNK
EPAD = EPW * NW
NPAD = 10240
RPT = NPAD // NS
BR = 256


def _mesh():
    return plsc.VectorSubcoreMesh(core_axis_name="c", subcore_axis_name="s")


def _sc_degree_body(dst_hbm, out_hbm, dst_v, ones_v, deg_sh):
    cid = lax.axis_index("c")
    sid = lax.axis_index("s")
    wid = sid * NC + cid
    base = sid * RPT

    def zrow(r, _):
        ones_v[r] = jnp.zeros((16,), jnp.float32)
        return 0
    lax.fori_loop(0, CHUNK, zrow, 0)
    for k in range(RPT // CHUNK):
        pltpu.sync_copy(ones_v, deg_sh.at[pl.ds(base + k * CHUNK, CHUNK)])

    def orow(r, _):
        ones_v[r] = jnp.ones((16,), jnp.float32)
        return 0
    lax.fori_loop(0, CHUNK, orow, 0)
    plsc.subcore_barrier()

    pltpu.sync_copy(dst_hbm.at[wid], dst_v)

    def step(j, _):
        pltpu.sync_copy(ones_v, deg_sh.at[dst_v.at[j]], add=True)
        return 0
    lax.fori_loop(0, NCHUNK, step, 0)

    plsc.subcore_barrier()
    pltpu.sync_copy(deg_sh.at[pl.ds(base, RPT)], out_hbm.at[cid, pl.ds(base, RPT)])


def _sc_degree(dst3):
    return pl.kernel(
        _sc_degree_body,
        out_type=jax.ShapeDtypeStruct((NC, NPAD, 16), jnp.float32),
        mesh=_mesh(),
        scratch_types=[
            pltpu.VMEM((NCHUNK, CHUNK), jnp.int32),
            pltpu.VMEM((CHUNK, 16), jnp.float32),
            pltpu.VMEM_SHARED((NPAD, 16), jnp.float32),
        ],
    )(dst3)


def _sc_edges_body(g_hbm, src_hbm, dst_hbm, out_hbm, src_v, dst_v, rows_v,
                   acc_sh, sem):
    cid = lax.axis_index("c")
    sid = lax.axis_index("s")
    wid = sid * NC + cid
    base = sid * RPT

    def zrow(r, _):
        def zcol(cc, _):
            rows_v[r, pl.ds(cc * 16, 16)] = jnp.zeros((16,), jnp.float32)
            return 0
        lax.fori_loop(0, H // 16, zcol, 0)
        return 0
    lax.fori_loop(0, CHUNK, zrow, 0)
    for k in range(RPT // CHUNK):
        pltpu.sync_copy(rows_v, acc_sh.at[pl.ds(base + k * CHUNK, CHUNK)])
    plsc.subcore_barrier()

    pltpu.sync_copy(src_hbm.at[wid], src_v)
    pltpu.sync_copy(dst_hbm.at[wid], dst_v)

    def step(j, _):
        pltpu.async_copy(g_hbm.at[src_v.at[j]], rows_v, sem).wait()
        pltpu.sync_copy(rows_v, acc_sh.at[dst_v.at[j]], add=True)
        return 0
    lax.fori_loop(0, NCHUNK, step, 0)

    plsc.subcore_barrier()
    for k in range(RPT // CHUNK):
        pltpu.sync_copy(
            acc_sh.at[pl.ds(base + k * CHUNK, CHUNK)],
            out_hbm.at[cid, pl.ds(base + k * CHUNK, CHUNK)],
        )


def _sc_edges(g, src3, dst3):
    return pl.kernel(
        _sc_edges_body,
        out_type=jax.ShapeDtypeStruct((NC, NPAD, H), jnp.float32),
        mesh=_mesh(),
        scratch_types=[
            pltpu.VMEM((NCHUNK, CHUNK), jnp.int32),
            pltpu.VMEM((NCHUNK, CHUNK), jnp.int32),
            pltpu.VMEM((CHUNK, H), jnp.float32),
            pltpu.VMEM_SHARED((NPAD, H), jnp.float32),
            pltpu.SemaphoreType.DMA,
        ],
    )(g, src3, dst3)


def _tc_prep_body(x_ref, wt_ref, deg_ref, g_ref, dinv_ref):
    h = jnp.dot(x_ref[...], wt_ref[...], preferred_element_type=jnp.float32)
    d = deg_ref[0] + deg_ref[1]
    dinv = lax.rsqrt(d[:, 0:1] + 1.0)
    g_ref[...] = h * dinv
    dinv_ref[...] = jnp.broadcast_to(dinv, (BR, H))


def _tc_prep(x_p, wt, deg2):
    grid = (NPAD // BR,)
    return pl.pallas_call(
        _tc_prep_body,
        grid=grid,
        in_specs=[
            pl.BlockSpec((BR, D), lambda i: (i, 0)),
            pl.BlockSpec((D, H), lambda i: (0, 0)),
            pl.BlockSpec((NC, BR, 16), lambda i: (0, i, 0)),
        ],
        out_specs=[
            pl.BlockSpec((BR, H), lambda i: (i, 0)),
            pl.BlockSpec((BR, H), lambda i: (i, 0)),
        ],
        out_shape=[
            jax.ShapeDtypeStruct((NPAD, H), jnp.float32),
            jax.ShapeDtypeStruct((NPAD, H), jnp.float32),
        ],
    )(x_p, wt, deg2)


def _tc_final_body(acc_ref, g_ref, dinv_ref, gcnb_ref, wih_ref, bih_ref,
                   bhh_ref, linwt_ref, linb_ref, out_ref):
    a = acc_ref[0] + acc_ref[1] + g_ref[...]
    z = jnp.maximum(a * dinv_ref[...] + gcnb_ref[...], 0.0)
    for l in range(L):
        gates = jnp.dot(z, wih_ref[l], preferred_element_type=jnp.float32)
        gates = gates + bih_ref[l] + bhh_ref[l]
        i_ = gates[:, 0:H]
        gg = gates[:, 2 * H:3 * H]
        o_ = gates[:, 3 * H:4 * H]
        c = jax.nn.sigmoid(i_) * jnp.tanh(gg)
        z = jax.nn.sigmoid(o_) * jnp.tanh(c)
    out_ref[...] = (
        jnp.dot(z, linwt_ref[...], preferred_element_type=jnp.float32)
        + linb_ref[...]
    )


def _tc_final(acc2, g, dinv2d, gcn_b2, wih_t, b_ih, b_hh, lin_wt, lin_b2):
    grid = (NPAD // BR,)
    return pl.pallas_call(
        _tc_final_body,
        grid=grid,
        in_specs=[
            pl.BlockSpec((NC, BR, H), lambda i: (0, i, 0)),
            pl.BlockSpec((BR, H), lambda i: (i, 0)),
            pl.BlockSpec((BR, H), lambda i: (i, 0)),
            pl.BlockSpec((1, H), lambda i: (0, 0)),
            pl.BlockSpec((L, H, 4 * H), lambda i: (0, 0, 0)),
            pl.BlockSpec((L, 4 * H), lambda i: (0, 0)),
            pl.BlockSpec((L, 4 * H), lambda i: (0, 0)),
            pl.BlockSpec((H, OUT), lambda i: (0, 0)),
            pl.BlockSpec((1, OUT), lambda i: (0, 0)),
        ],
        out_specs=pl.BlockSpec((BR, OUT), lambda i: (i, 0)),
        out_shape=jax.ShapeDtypeStruct((NPAD, OUT), jnp.float32),
    )(acc2, g, dinv2d, gcn_b2, wih_t, b_ih, b_hh, lin_wt, lin_b2)


@jax.jit
def kernel(x, edge_index, gcn_W, gcn_b, W_ih, W_hh, b_ih, b_hh, lin_W, lin_b):
    del W_hh

    x_p = jnp.zeros((NPAD, D), jnp.float32).at[:N].set(x)
    src = edge_index[0]
    dst = edge_index[1]
    pad = EPAD - src.shape[0]
    src3 = jnp.concatenate([src, jnp.zeros((pad,), jnp.int32)]).reshape(
        NW, NCHUNK, CHUNK)
    dst3 = jnp.concatenate([dst, jnp.full((pad,), N, jnp.int32)]).reshape(
        NW, NCHUNK, CHUNK)
    wt = gcn_W.T
    wih_t = W_ih.transpose(0, 2, 1)
    lin_wt = lin_W.T
    gcn_b2 = gcn_b.reshape(1, H)
    lin_b2 = lin_b.reshape(1, OUT)

    deg2 = _sc_degree(dst3)
    g, dinv2d = _tc_prep(x_p, wt, deg2)
    acc2 = _sc_edges(g, src3, dst3)
    out = _tc_final(acc2, g, dinv2d, gcn_b2, wih_t, b_ih, b_hh, lin_wt,
                    lin_b2)
    return out[:N]
